# baseline (speedup 1.0000x reference)
import functools

import jax
import jax.numpy as jnp
from jax import lax
from jax.experimental import pallas as pl
from jax.experimental.pallas import tpu as pltpu
from jax.experimental.pallas import tpu_sc as plsc

_NC = 2
_NS = 16
_CH = 128
_L = 16


def _rsqrt(x):
    i = lax.bitcast_convert_type(x, jnp.int32)
    i = 0x5F3759DF - (i >> 1)
    y = lax.bitcast_convert_type(i, jnp.float32)
    hx = 0.5 * x
    for _ in range(3):
        y = y * (1.5 - hx * y * y)
    return y


def _acos(a):
    s = 1.0 - a
    sq = s * _rsqrt(jnp.maximum(s, 1e-30))
    return sq * (
        1.5707288 + a * (-0.2121144 + a * (0.0742610 + a * (-0.0187293)))
    )


def _sc_fused(table2d, idx1d, b, t_steps, d):
    nw = _NC * _NS
    bw = b // nw
    per_w = bw * t_steps
    n_ch = per_w // _CH
    n_grp = bw // _L
    mesh = plsc.VectorSubcoreMesh(core_axis_name="c", subcore_axis_name="s")

    @functools.partial(
        pl.kernel,
        mesh=mesh,
        out_type=(
            jax.ShapeDtypeStruct((b * d,), jnp.float32),
            jax.ShapeDtypeStruct((b * t_steps,), jnp.float32),
        ),
        scratch_types=[
            pltpu.VMEM((per_w,), jnp.int32),
            pltpu.VMEM((per_w, d), jnp.float32),
            pltpu.VMEM((bw * d,), jnp.float32),
            pltpu.VMEM((per_w,), jnp.float32),
            pltpu.SemaphoreType.DMA,
        ],
        compiler_params=pltpu.CompilerParams(
            use_tc_tiling_on_sc=False, needs_layout_passes=False),
    )
    def gk(table_hbm, idx_hbm, c_hbm, s_hbm, idx_v, rows_v, cbuf, sigbuf,
           sem):
        wid = lax.axis_index("s") * _NC + lax.axis_index("c")
        rbase = wid * per_w
        pltpu.sync_copy(idx_hbm.at[pl.ds(rbase, per_w)], idx_v)
        copies = [
            pltpu.async_copy(
                table_hbm.at[idx_v.at[pl.ds(j * _CH, _CH)]],
                rows_v.at[pl.ds(j * _CH, _CH)],
                sem,
            )
            for j in range(n_ch)
        ]
        for cc in copies:
            cc.wait()

        ar16 = lax.iota(jnp.int32, _L)
        zeros = jnp.zeros((_L,), jnp.float32)
        ones = jnp.ones((_L,), jnp.float32)
        zi = jnp.zeros((_L,), jnp.int32)

        def zero_sig(i, carry):
            sigbuf[pl.ds(i * _L, _L)] = zeros
            return carry

        lax.fori_loop(0, per_w // _L, zero_sig, 0)

        def g_loop(g, carry):
            b16 = g * _L + ar16

            def m_loop(m, carry_m):
                col = zi + m * 4

                def t_loop(t, C):
                    Cw, Cx, Cy, Cz = C
                    rowv = b16 * t_steps + t
                    gw = plsc.load_gather(rows_v, [rowv, col])
                    gx = plsc.load_gather(rows_v, [rowv, col + 1])
                    gy = plsc.load_gather(rows_v, [rowv, col + 2])
                    gz = plsc.load_gather(rows_v, [rowv, col + 3])
                    yn = _rsqrt(gw * gw + gx * gx + gy * gy + gz * gz)
                    gw, gx, gy, gz = gw * yn, gx * yn, gy * yn, gz * yn
                    w = Cw * gw - Cx * gx - Cy * gy - Cz * gz
                    x = Cw * gx + Cx * gw + Cy * gz - Cz * gy
                    y = Cw * gy - Cx * gz + Cy * gw + Cz * gx
                    z = Cw * gz + Cx * gy - Cy * gx + Cz * gw
                    yc = _rsqrt(w * w + x * x + y * y + z * z)
                    aw = jnp.minimum(jnp.abs(w) * yc, 1.0 - 1e-7)
                    pf = _acos(aw)
                    sv = plsc.load_gather(sigbuf, [rowv])
                    plsc.store_scatter(sigbuf, [rowv], sv + pf)
                    return (w, x, y, z)

                Cw, Cx, Cy, Cz = lax.fori_loop(
                    0, t_steps, t_loop, (ones, zeros, zeros, zeros))
                yc = _rsqrt(Cw * Cw + Cx * Cx + Cy * Cy + Cz * Cz)
                cbase = b16 * d + m * 4
                plsc.store_scatter(cbuf, [cbase], Cw * yc)
                plsc.store_scatter(cbuf, [cbase + 1], Cx * yc)
                plsc.store_scatter(cbuf, [cbase + 2], Cy * yc)
                plsc.store_scatter(cbuf, [cbase + 3], Cz * yc)
                return carry_m

            lax.fori_loop(0, d // 4, m_loop, 0)
            return carry

        lax.fori_loop(0, n_grp, g_loop, 0)

        def scale_sig(i, carry):
            sl = pl.ds(i * _L, _L)
            sigbuf[sl] = sigbuf[sl] * 0.125
            return carry

        lax.fori_loop(0, per_w // _L, scale_sig, 0)

        pltpu.sync_copy(cbuf, c_hbm.at[pl.ds(wid * bw * d, bw * d)])
        pltpu.sync_copy(sigbuf, s_hbm.at[pl.ds(rbase, per_w)])

    return gk(table2d, idx1d)


def kernel(tokens, embed):
    b, t_steps = tokens.shape
    vocab, m_dim, four = embed.shape
    d = m_dim * four
    table2d = embed.reshape(vocab, d)
    idx1d = tokens.astype(jnp.int32).reshape(b * t_steps)
    c_flat, s_flat = _sc_fused(table2d, idx1d, b, t_steps, d)
    return c_flat.reshape(b, m_dim, four), s_flat.reshape(b, t_steps)


# speedup vs baseline: 1.1335x; 1.1335x over previous
import functools

import jax
import jax.numpy as jnp
from jax import lax
from jax.experimental import pallas as pl
from jax.experimental.pallas import tpu as pltpu
from jax.experimental.pallas import tpu_sc as plsc

_NC = 2
_NS = 16
_CH = 128


def _sc_gather(table2d, idx1d, bt, d):
    nw = _NC * _NS
    per_w = bt // nw
    n_ch = per_w // _CH
    mesh = plsc.VectorSubcoreMesh(core_axis_name="c", subcore_axis_name="s")

    @functools.partial(
        pl.kernel,
        mesh=mesh,
        out_type=jax.ShapeDtypeStruct((bt, d), jnp.float32),
        scratch_types=[
            pltpu.VMEM((per_w,), jnp.int32),
            pltpu.VMEM((per_w, d), jnp.float32),
            pltpu.SemaphoreType.DMA,
        ],
        compiler_params=pltpu.CompilerParams(use_tc_tiling_on_sc=False),
    )
    def gk(table_hbm, idx_hbm, out_hbm, idx_v, rows_v, sem):
        wid = lax.axis_index("s") * _NC + lax.axis_index("c")
        base = wid * per_w
        pltpu.sync_copy(idx_hbm.at[pl.ds(base, per_w)], idx_v)
        copies = []
        for j in range(n_ch):
            copies.append(
                pltpu.async_copy(
                    table_hbm.at[idx_v.at[pl.ds(j * _CH, _CH)]],
                    rows_v.at[pl.ds(j * _CH, _CH)],
                    sem,
                )
            )
        for c in copies:
            c.wait()
        pltpu.sync_copy(rows_v, out_hbm.at[pl.ds(base, per_w)])

    return gk(table2d, idx1d)


def _tc_compose(g2, b, t_steps):
    blk = 512
    grid = b // blk

    def body(g_ref, c_ref, s_ref):
        r = lax.broadcasted_iota(jnp.int32, (32, 32), 0)
        j = lax.broadcasted_iota(jnp.int32, (32, 32), 1)
        P = jnp.where(j == 4 * (r % 8) + r // 8, 1.0, 0.0).astype(jnp.float32)
        dn_ext = (((1,), (1,)), ((), ()))
        dn_sto = (((0,), (0,)), ((), ()))
        hi = lax.Precision.HIGHEST

        Cw = jnp.ones((8, blk), jnp.float32)
        Cx = jnp.zeros((8, blk), jnp.float32)
        Cy = jnp.zeros((8, blk), jnp.float32)
        Cz = jnp.zeros((8, blk), jnp.float32)
        for t in range(t_steps):
            gt = g_ref[:, t * 32:(t + 1) * 32]
            R = lax.dot_general(P, gt, dn_ext, precision=hi)
            Gw, Gx, Gy, Gz = R[0:8], R[8:16], R[16:24], R[24:32]
            n2 = Gw * Gw + Gx * Gx + Gy * Gy + Gz * Gz
            inv = 1.0 / jnp.maximum(jnp.sqrt(n2), 1e-12)
            Gw, Gx, Gy, Gz = Gw * inv, Gx * inv, Gy * inv, Gz * inv
            w = Cw * Gw - Cx * Gx - Cy * Gy - Cz * Gz
            x = Cw * Gx + Cx * Gw + Cy * Gz - Cz * Gy
            y = Cw * Gy - Cx * Gz + Cy * Gw + Cz * Gx
            z = Cw * Gz + Cx * Gy - Cy * Gx + Cz * Gw
            n2c = w * w + x * x + y * y + z * z
            invc = 1.0 / jnp.maximum(jnp.sqrt(n2c), 1e-12)
            Cw, Cx, Cy, Cz = w * invc, x * invc, y * invc, z * invc
            a = jnp.minimum(jnp.abs(Cw), 1.0 - 1e-7)
            pf = jnp.sqrt(1.0 - a) * (
                1.5707288
                + a * (-0.2121144 + a * (0.0742610 + a * (-0.0187293)))
            )
            s_ref[:, t] = jnp.mean(pf, axis=0)
        Cstack = jnp.concatenate([Cw, Cx, Cy, Cz], axis=0)
        c_ref[...] = lax.dot_general(Cstack, P, dn_sto, precision=hi)

    return pl.pallas_call(
        body,
        grid=(grid,),
        in_specs=[pl.BlockSpec((blk, t_steps * 32), lambda i: (i, 0))],
        out_specs=[
            pl.BlockSpec((blk, 32), lambda i: (i, 0)),
            pl.BlockSpec((blk, t_steps), lambda i: (i, 0)),
        ],
        out_shape=[
            jax.ShapeDtypeStruct((b, 32), jnp.float32),
            jax.ShapeDtypeStruct((b, t_steps), jnp.float32),
        ],
    )(g2)


def kernel(tokens, embed):
    b, t_steps = tokens.shape
    vocab, m, four = embed.shape
    d = m * four
    bt = b * t_steps
    table2d = embed.reshape(vocab, d)
    idx1d = tokens.astype(jnp.int32).reshape(bt)
    gathered = _sc_gather(table2d, idx1d, bt, d)
    g2 = gathered.reshape(b, t_steps * d)
    c2, sigmas = _tc_compose(g2, b, t_steps)
    return c2.reshape(b, m, four), sigmas
